# baseline (speedup 1.0000x reference)
import functools

import jax
import jax.numpy as jnp
from jax import lax
from jax.experimental import pallas as pl
from jax.experimental.pallas import tpu as pltpu
from jax.experimental.pallas import tpu_sc as plsc

B = 4096
H = 50
D = 64
HP = 64
V = 1000000
NC = 2
NS = 16
NW = NC * NS
BPW = B // NW
CB = 2
RPC = CB * H
NCHUNK = BPW // CB
CPG = 16 // CB
NBUF = 4


def _sc_body(pair_hbm, woff_hbm, mask_hbm, table_hbm, out_hbm,
             idx_v, woff_v, mask_v, rows_b0, rows_b1, rows_b2, rows_b3,
             scores_v, sem0, sem1, sem2, sem3):
    rows_b = [rows_b0, rows_b1, rows_b2, rows_b3]
    sems = [sem0, sem1, sem2, sem3]
    wid = lax.axis_index("s") * NC + lax.axis_index("c")
    pltpu.sync_copy(pair_hbm.at[pl.ds(wid * NCHUNK, NCHUNK)], idx_v)
    pltpu.sync_copy(woff_hbm.at[pl.ds(wid * BPW, BPW)], woff_v)
    pltpu.sync_copy(mask_hbm.at[pl.ds(wid * BPW, BPW)], mask_v)
    lanes = lax.iota(jnp.int32, 16)

    for b in range(NBUF):
        pltpu.async_copy(table_hbm.at[idx_v.at[b]], rows_b[b], sems[b])

    def compute_chunk(j, bb, carry):
        dot_vec, dn_vec = carry
        rows_v = rows_b[bb]
        pltpu.make_async_copy(
            table_hbm.at[idx_v.at[j]], rows_v, sems[bb]).wait()
        off = (j % CPG) * CB
        for t in range(CB):
            b_local = j * CB + t
            roff = t * H
            mv = [mask_v[b_local, pl.ds(16 * k, 16)] for k in range(HP // 16)]
            wv = [woff_v[b_local, pl.ds(16 * k, 16)] for k in range(HP // 16)]
            ms = [mv[k][i] for k in range(4) for i in range(16)]
            ws = [wv[k][i] for k in range(4) for i in range(16)]
            acc = [jnp.zeros((16,), jnp.float32) for _ in range(4)]
            dn = jnp.float32(0.0)
            for l in range(1, H):
                m = ms[l]
                o = ws[l]
                dn = dn + m
                r = roff + l
                for d in range(4):
                    acc[d] = acc[d] + m * rows_v[r, pl.ds(o + 16 * d, 16)]
            pd = jnp.zeros((16,), jnp.float32)
            o0 = ws[0]
            for d in range(4):
                pd = pd + rows_v[roff, pl.ds(o0 + 16 * d, 16)] * acc[d]
            dot = jnp.float32(0.0)
            for i in range(16):
                dot = dot + pd[i]
            here = lanes == off + t
            dot_vec = jnp.where(here, dot, dot_vec)
            dn_vec = jnp.where(here, dn, dn_vec)

        @pl.when(j + NBUF < NCHUNK)
        def _():
            pltpu.async_copy(
                table_hbm.at[idx_v.at[j + NBUF]], rows_v, sems[bb])

        done = (j % CPG) == (CPG - 1)

        @pl.when(done)
        def _():
            scores_v[pl.ds((j // CPG) * 16, 16)] = (
                dot_vec / jnp.maximum(dn_vec, 1.0))

        z = jnp.zeros((16,), jnp.float32)
        return (jnp.where(done, z, dot_vec), jnp.where(done, z, dn_vec))

    def ring_body(g, carry):
        for bb in range(NBUF):
            carry = compute_chunk(g * NBUF + bb, bb, carry)
        return carry

    z0 = jnp.zeros((16,), jnp.float32)
    lax.fori_loop(0, NCHUNK // NBUF, ring_body, (z0, z0))
    pltpu.sync_copy(scores_v, out_hbm.at[pl.ds(wid * BPW, BPW)])


_sc_call = functools.partial(
    pl.kernel,
    out_type=jax.ShapeDtypeStruct((B,), jnp.float32),
    mesh=plsc.VectorSubcoreMesh(core_axis_name="c", subcore_axis_name="s"),
    scratch_types=[
        pltpu.VMEM((NCHUNK, RPC), jnp.int32),
        pltpu.VMEM((BPW, HP), jnp.int32),
        pltpu.VMEM((BPW, HP), jnp.float32),
        pltpu.VMEM((RPC, 2 * D), jnp.float32),
        pltpu.VMEM((RPC, 2 * D), jnp.float32),
        pltpu.VMEM((RPC, 2 * D), jnp.float32),
        pltpu.VMEM((RPC, 2 * D), jnp.float32),
        pltpu.VMEM((BPW,), jnp.float32),
        pltpu.SemaphoreType.DMA,
        pltpu.SemaphoreType.DMA,
        pltpu.SemaphoreType.DMA,
        pltpu.SemaphoreType.DMA,
    ],
)(_sc_body)


def kernel(inds, mask, table):
    inds32 = inds.astype(jnp.int32)
    pair = (inds32 >> 1).reshape(NW * NCHUNK, RPC)
    woff = jnp.pad((inds32 & 1) << 6, ((0, 0), (0, HP - H)))
    mask_p = jnp.pad(mask, ((0, 0), (0, HP - H)))
    table2 = table.reshape(V // 2, 2 * D)
    return _sc_call(pair, woff, mask_p, table2)


# speedup vs baseline: 1.0103x; 1.0103x over previous
import functools

import jax
import jax.numpy as jnp
from jax import lax
from jax.experimental import pallas as pl
from jax.experimental.pallas import tpu as pltpu
from jax.experimental.pallas import tpu_sc as plsc

B = 4096
H = 50
D = 64
HP = 64
NC = 2
NS = 16
NW = NC * NS
BPW = B // NW
CB = 1
RPC = CB * H
NCHUNK = BPW // CB
CPG = 16 // CB
NBUF = 2
RB = 8 * (RPC + 1)


def _fire(table_hbm, idx_c, b, rows_v, sem):
    ivs = [idx_c[b, pl.ds(16 * k, 16)] for k in range(3)]
    ivs.append(idx_c[b, pl.ds(RPC - 16, 16)])
    for s in range(RPC):
        k, i = (s // 16, s % 16) if s < 48 else (3, s - (RPC - 16))
        r = ivs[k][i]
        low = r & 7
        base = pl.multiple_of(r - low, 8)
        pltpu.async_copy(
            table_hbm.at[pl.ds(base, 8)],
            rows_v.at[pl.ds(8 * (s + 1) - low, 8)],
            sem)


def _drain(table_hbm, rows_v, sem):
    pltpu.make_async_copy(
        table_hbm.at[pl.ds(0, 8 * RPC)], rows_v.at[pl.ds(0, 8 * RPC)],
        sem).wait()


def _sc_body(inds_hbm, mask_hbm, table_hbm, out_hbm,
             idx_c, mask_v, rows_b0, rows_b1, scores_v,
             sem0, sem1, isem0, isem1):
    rows_b = [rows_b0, rows_b1]
    sems = [sem0, sem1]
    isems = [isem0, isem1]
    wid = lax.axis_index("s") * NC + lax.axis_index("c")
    row0 = wid * NCHUNK
    pltpu.sync_copy(mask_hbm.at[pl.ds(wid * BPW, BPW)], mask_v)
    lanes = lax.iota(jnp.int32, 16)

    for b in range(NBUF):
        pltpu.sync_copy(inds_hbm.at[pl.ds(row0 + b, 1)],
                        idx_c.at[pl.ds(b, 1)])
        _fire(table_hbm, idx_c, b, rows_b[b], sems[b])
    for b in range(NBUF):
        pltpu.async_copy(inds_hbm.at[pl.ds(row0 + NBUF + b, 1)],
                         idx_c.at[pl.ds(b, 1)], isems[b])

    def compute_chunk(j, bb, carry):
        dot_vec, dn_vec = carry
        rows_v = rows_b[bb]
        _drain(table_hbm, rows_v, sems[bb])
        off = (j % CPG) * CB
        for t in range(CB):
            b_local = j * CB + t
            mv = [mask_v[b_local, pl.ds(16 * k, 16)] for k in range(HP // 16)]
            ms = [mv[k][i] for k in range(4) for i in range(16)]
            acc = [jnp.zeros((16,), jnp.float32) for _ in range(4)]
            dn = jnp.float32(0.0)
            for l in range(1, H):
                m = ms[l]
                dn = dn + m
                r = 8 * (t * H + l + 1)
                for d in range(4):
                    acc[d] = acc[d] + m * rows_v[r, pl.ds(16 * d, 16)]
            pd = jnp.zeros((16,), jnp.float32)
            rid = 8 * (t * H + 1)
            for d in range(4):
                pd = pd + rows_v[rid, pl.ds(16 * d, 16)] * acc[d]
            dot = jnp.float32(0.0)
            for i in range(16):
                dot = dot + pd[i]
            here = lanes == off + t
            dot_vec = jnp.where(here, dot, dot_vec)
            dn_vec = jnp.where(here, dn, dn_vec)

        @pl.when(j + NBUF < NCHUNK)
        def _():
            pltpu.make_async_copy(
                inds_hbm.at[pl.ds(row0, 1)], idx_c.at[pl.ds(bb, 1)],
                isems[bb]).wait()
            _fire(table_hbm, idx_c, bb, rows_v, sems[bb])

        @pl.when(j + 2 * NBUF < NCHUNK)
        def _():
            pltpu.async_copy(inds_hbm.at[pl.ds(row0 + j + 2 * NBUF, 1)],
                             idx_c.at[pl.ds(bb, 1)], isems[bb])

        done = (j % CPG) == (CPG - 1)

        @pl.when(done)
        def _():
            scores_v[pl.ds((j // CPG) * 16, 16)] = (
                dot_vec / jnp.maximum(dn_vec, 1.0))

        z = jnp.zeros((16,), jnp.float32)
        return (jnp.where(done, z, dot_vec), jnp.where(done, z, dn_vec))

    def ring_body(g, carry):
        for bb in range(NBUF):
            carry = compute_chunk(g * NBUF + bb, bb, carry)
        return carry

    z0 = jnp.zeros((16,), jnp.float32)
    lax.fori_loop(0, NCHUNK // NBUF, ring_body, (z0, z0))
    pltpu.sync_copy(scores_v, out_hbm.at[pl.ds(wid * BPW, BPW)])


_sc_call = functools.partial(
    pl.kernel,
    out_type=jax.ShapeDtypeStruct((B,), jnp.float32),
    mesh=plsc.VectorSubcoreMesh(core_axis_name="c", subcore_axis_name="s"),
    scratch_types=[
        pltpu.VMEM((NBUF, RPC), jnp.int32),
        pltpu.VMEM((BPW, HP), jnp.float32),
        pltpu.VMEM((RB, D), jnp.float32),
        pltpu.VMEM((RB, D), jnp.float32),
        pltpu.VMEM((BPW,), jnp.float32),
        pltpu.SemaphoreType.DMA,
        pltpu.SemaphoreType.DMA,
        pltpu.SemaphoreType.DMA,
        pltpu.SemaphoreType.DMA,
    ],
)(_sc_body)


def kernel(inds, mask, table):
    inds2 = inds.astype(jnp.int32).reshape(NW * NCHUNK, RPC)
    mask_p = jnp.pad(mask, ((0, 0), (0, HP - H)))
    return _sc_call(inds2, mask_p, table)


# speedup vs baseline: 1.0381x; 1.0275x over previous
import functools

import jax
import jax.numpy as jnp
from jax import lax
from jax.experimental import pallas as pl
from jax.experimental.pallas import tpu as pltpu
from jax.experimental.pallas import tpu_sc as plsc

B = 4096
H = 50
D = 64
HP = 64
NC = 2
NS = 16
NW = NC * NS
BPW = B // NW
CB = 2
NCHUNK = BPW // CB
CPG = 16 // CB
NBUF = 4


def _sc_body(inds_hbm, mask_hbm, table_hbm, out_hbm,
             idx_v, mask_v, rows_b0, rows_b1, rows_b2, rows_b3, scores_v,
             sem0, sem1, sem2, sem3):
    rows_b = [rows_b0, rows_b1, rows_b2, rows_b3]
    sems = [sem0, sem1, sem2, sem3]
    wid = lax.axis_index("s") * NC + lax.axis_index("c")
    pltpu.sync_copy(inds_hbm.at[pl.ds(wid * NCHUNK, NCHUNK)], idx_v)
    pltpu.sync_copy(mask_hbm.at[pl.ds(wid * BPW, BPW)], mask_v)
    lanes = lax.iota(jnp.int32, 16)

    for b in range(NBUF):
        pltpu.async_copy(table_hbm.at[idx_v.at[b]], rows_b[b], sems[b])

    def compute_chunk(j, bb, carry):
        dot_vec, dn_vec = carry
        rows_v = rows_b[bb]
        pltpu.make_async_copy(
            table_hbm.at[idx_v.at[j]], rows_v, sems[bb]).wait()
        off = (j % CPG) * CB
        for t in range(CB):
            b_local = j * CB + t
            roff = t * H
            mv = [mask_v[b_local, pl.ds(16 * k, 16)] for k in range(HP // 16)]
            ms = [mv[k][i] for k in range(4) for i in range(16)]
            acc = [jnp.zeros((16,), jnp.float32) for _ in range(4)]
            dn = jnp.float32(0.0)
            for l in range(1, H):
                m = ms[l]
                dn = dn + m
                r = roff + l
                for d in range(4):
                    acc[d] = acc[d] + m * rows_v[r, pl.ds(16 * d, 16)]
            pd = jnp.zeros((16,), jnp.float32)
            for d in range(4):
                pd = pd + rows_v[roff, pl.ds(16 * d, 16)] * acc[d]
            dot = jnp.float32(0.0)
            for i in range(16):
                dot = dot + pd[i]
            here = lanes == off + t
            dot_vec = jnp.where(here, dot, dot_vec)
            dn_vec = jnp.where(here, dn, dn_vec)

        @pl.when(j + NBUF < NCHUNK)
        def _():
            pltpu.async_copy(
                table_hbm.at[idx_v.at[j + NBUF]], rows_v, sems[bb])

        done = (j % CPG) == (CPG - 1)

        @pl.when(done)
        def _():
            scores_v[pl.ds((j // CPG) * 16, 16)] = (
                dot_vec / jnp.maximum(dn_vec, 1.0))

        z = jnp.zeros((16,), jnp.float32)
        return (jnp.where(done, z, dot_vec), jnp.where(done, z, dn_vec))

    def ring_body(g, carry):
        for bb in range(NBUF):
            carry = compute_chunk(g * NBUF + bb, bb, carry)
        return carry

    z0 = jnp.zeros((16,), jnp.float32)
    lax.fori_loop(0, NCHUNK // NBUF, ring_body, (z0, z0))
    pltpu.sync_copy(scores_v, out_hbm.at[pl.ds(wid * BPW, BPW)])


_sc_call = functools.partial(
    pl.kernel,
    out_type=jax.ShapeDtypeStruct((B,), jnp.float32),
    mesh=plsc.VectorSubcoreMesh(core_axis_name="c", subcore_axis_name="s"),
    compiler_params=pltpu.CompilerParams(use_tc_tiling_on_sc=False),
    scratch_types=[
        pltpu.VMEM((NCHUNK, CB * H), jnp.int32),
        pltpu.VMEM((BPW, HP), jnp.float32),
        pltpu.VMEM((CB * H, D), jnp.float32),
        pltpu.VMEM((CB * H, D), jnp.float32),
        pltpu.VMEM((CB * H, D), jnp.float32),
        pltpu.VMEM((CB * H, D), jnp.float32),
        pltpu.VMEM((BPW,), jnp.float32),
        pltpu.SemaphoreType.DMA,
        pltpu.SemaphoreType.DMA,
        pltpu.SemaphoreType.DMA,
        pltpu.SemaphoreType.DMA,
    ],
)(_sc_body)


def kernel(inds, mask, table):
    inds2 = inds.astype(jnp.int32).reshape(NW * NCHUNK, CB * H)
    mask_p = jnp.pad(mask, ((0, 0), (0, HP - H)))
    return _sc_call(inds2, mask_p, table)
